# baseline (speedup 1.0000x reference)
import functools

import jax
import jax.numpy as jnp
from jax import lax
from jax.experimental import pallas as pl
from jax.experimental.pallas import tpu as pltpu
from jax.experimental.pallas import tpu_sc as plsc

NC = 2
NS = 16
NW = NC * NS
LANE = 128
SUB = 8
NBUF = 8


@functools.lru_cache(maxsize=None)
def _build(batch, seq, dim):
    bt = batch // LANE
    assert bt == NW and seq % SUB == 0
    sg = seq // SUB
    lg = LANE // 16

    mesh = plsc.VectorSubcoreMesh(core_axis_name="c", subcore_axis_name="s")

    @functools.partial(
        pl.kernel,
        mesh=mesh,
        out_type=jax.ShapeDtypeStruct((batch, seq * 2 * dim), jnp.float32),
        compiler_params=pltpu.CompilerParams(
            use_tc_tiling_on_sc=False, needs_layout_passes=False
        ),
        scratch_types=[
            pltpu.VMEM((sg, SUB, LANE), jnp.int32),
            pltpu.VMEM((NBUF, LANE), jnp.int32),
            pltpu.VMEM((NBUF, LANE, dim), jnp.float32),
            [pltpu.SemaphoreType.DMA] * NBUF,
            [pltpu.SemaphoreType.DMA] * NBUF,
        ],
    )
    def emb(ids_hbm, table_hbm, out_hbm, idx_v, stage, bufs, gsem, wsem):
        w = lax.axis_index("s") * NC + lax.axis_index("c")

        pltpu.sync_copy(ids_hbm.at[pl.ds(0, 1), w], idx_v.at[pl.ds(0, 1)])

        def stage_and_fire(g, r, slot):
            for l in range(lg):
                iv = idx_v[g, r, pl.ds(16 * l, 16)]
                stage[slot, pl.ds(16 * l, 16)] = iv + iv
            pltpu.make_async_copy(
                table_hbm.at[stage.at[slot]], bufs.at[slot], gsem[slot]
            ).start()

        def wait_gather(slot):
            pltpu.make_async_copy(
                table_hbm.at[stage.at[slot]], bufs.at[slot], gsem[slot]
            ).wait()

        def out_slice(g, r):
            s = g * SUB + r
            return out_hbm.at[pl.ds(w * LANE, LANE), pl.ds(s * 2 * dim, dim)]

        def fire_write(g, r, slot):
            pltpu.make_async_copy(
                bufs.at[slot], out_slice(g, r), wsem[slot]
            ).start()

        def wait_write(g, r, slot):
            pltpu.make_async_copy(
                bufs.at[slot], out_slice(g, r), wsem[slot]
            ).wait()

        for k in range(NBUF - 1):
            stage_and_fire(0, k, k)
        pltpu.sync_copy(
            ids_hbm.at[pl.ds(1, sg - 1), w], idx_v.at[pl.ds(1, sg - 1)]
        )

        def body(g, carry):
            for r in range(SUB):
                slot = r % NBUF
                nslot = (r + NBUF - 1) % NBUF
                nr = r + NBUF - 1
                if nr < SUB:
                    if r == 0:
                        @pl.when(g > 0)
                        def _():
                            wait_write(g, r, nslot)
                    else:
                        wait_write(g, r, nslot)
                    stage_and_fire(g, nr, nslot)
                else:
                    @pl.when(g < sg - 1)
                    def _():
                        wait_write(g, r, nslot)
                        stage_and_fire(g + 1, nr - SUB, nslot)
                wait_gather(slot)
                fire_write(g, r, slot)
            return carry

        lax.fori_loop(0, sg, body, 0)
        for k in range(NBUF):
            wait_write(sg - 1, k, k)

    return emb


def kernel(word_ids, word_emb_table):
    batch, seq = word_ids.shape
    vocab, dim = word_emb_table.shape
    ids_lin = (
        word_ids.astype(jnp.int32)
        .T.reshape(seq // SUB, SUB, batch // LANE, LANE)
        .transpose(0, 2, 1, 3)
    )
    table_rows = jnp.pad(word_emb_table, ((0, 0), (0, dim))).reshape(
        2 * vocab, dim
    )
    emb = _build(batch, seq, dim)
    out_pad = emb(ids_lin, table_rows).reshape(batch, seq, 2 * dim)
    return out_pad[:, :, :dim]
